# baseline (speedup 1.0000x reference)
import functools

import jax
import jax.numpy as jnp
from jax.experimental import pallas as pl
from jax.experimental.pallas import tpu as pltpu

_EPS = 1e-5


def _emb_stats_kernel(x_ref, w_ref, xemb_ref, stats_ref, acc_ref):
    i = pl.program_id(0)
    n = pl.num_programs(0)
    xe = jnp.dot(x_ref[...], w_ref[...], preferred_element_type=jnp.float32)
    xemb_ref[...] = xe

    @pl.when(i == 0)
    def _():
        acc_ref[...] = jnp.zeros_like(acc_ref)

    acc_ref[0:1, :] += jnp.sum(xe, axis=0, keepdims=True)
    acc_ref[1:2, :] += jnp.sum(xe * xe, axis=0, keepdims=True)

    @pl.when(i == n - 1)
    def _():
        stats_ref[...] = acc_ref[...]


def _bn_relu_mm_kernel(xemb_ref, stats_ref, gamma_ref, beta_ref, w_ref,
                       sup_ref, *, n_rows):
    mu = stats_ref[0:1, :] / n_rows
    var = stats_ref[1:2, :] / n_rows - mu * mu
    a = gamma_ref[...] * jax.lax.rsqrt(var + _EPS)
    b = beta_ref[...] - mu * a
    h = jnp.maximum(xemb_ref[...] * a + b, 0.0)
    sup_ref[...] = jnp.dot(
        h, w_ref[...], preferred_element_type=jnp.float32
    ).astype(jnp.bfloat16)


def _spmm_stats_kernel(adj_ref, sup_ref, xgcn_ref, stats_ref, acc_ref):
    i = pl.program_id(0)
    n = pl.num_programs(0)
    a = adj_ref[...].astype(jnp.bfloat16)
    xg = jnp.dot(a, sup_ref[...], preferred_element_type=jnp.float32)
    xgcn_ref[...] = xg

    @pl.when(i == 0)
    def _():
        acc_ref[...] = jnp.zeros_like(acc_ref)

    acc_ref[0:1, :] += jnp.sum(xg, axis=0, keepdims=True)
    acc_ref[1:2, :] += jnp.sum(xg * xg, axis=0, keepdims=True)

    @pl.when(i == n - 1)
    def _():
        stats_ref[...] = acc_ref[...]


def _bn_kernel(xgcn_ref, stats_ref, gamma_ref, beta_ref, out_ref, *, n_rows):
    mu = stats_ref[0:1, :] / n_rows
    var = stats_ref[1:2, :] / n_rows - mu * mu
    a = gamma_ref[...] * jax.lax.rsqrt(var + _EPS)
    b = beta_ref[...] - mu * a
    out_ref[...] = xgcn_ref[...] * a + b


def kernel(x, adj, W_emb, gcn_weight, gamma_in, beta_in, gamma_local,
           beta_local):
    N, F = x.shape
    D = W_emb.shape[1]
    g_in = gamma_in.reshape(1, D)
    b_in = beta_in.reshape(1, D)
    g_loc = gamma_local.reshape(1, D)
    b_loc = beta_local.reshape(1, D)

    B1 = 1000
    x_emb, stats_in = pl.pallas_call(
        _emb_stats_kernel,
        grid=(N // B1,),
        in_specs=[
            pl.BlockSpec((B1, F), lambda i: (i, 0)),
            pl.BlockSpec((F, D), lambda i: (0, 0)),
        ],
        out_specs=[
            pl.BlockSpec((B1, D), lambda i: (i, 0)),
            pl.BlockSpec((2, D), lambda i: (0, 0)),
        ],
        out_shape=[
            jax.ShapeDtypeStruct((N, D), jnp.float32),
            jax.ShapeDtypeStruct((2, D), jnp.float32),
        ],
        scratch_shapes=[pltpu.VMEM((2, D), jnp.float32)],
        compiler_params=pltpu.CompilerParams(
            dimension_semantics=("arbitrary",)),
    )(x, W_emb)

    support = pl.pallas_call(
        functools.partial(_bn_relu_mm_kernel, n_rows=float(N)),
        grid=(N // B1,),
        in_specs=[
            pl.BlockSpec((B1, D), lambda i: (i, 0)),
            pl.BlockSpec((2, D), lambda i: (0, 0)),
            pl.BlockSpec((1, D), lambda i: (0, 0)),
            pl.BlockSpec((1, D), lambda i: (0, 0)),
            pl.BlockSpec((D, D), lambda i: (0, 0)),
        ],
        out_specs=pl.BlockSpec((B1, D), lambda i: (i, 0)),
        out_shape=jax.ShapeDtypeStruct((N, D), jnp.bfloat16),
    )(x_emb, stats_in, g_in, b_in, gcn_weight)

    B3 = 400
    x_gcn, stats_loc = pl.pallas_call(
        _spmm_stats_kernel,
        grid=(N // B3,),
        in_specs=[
            pl.BlockSpec((B3, N), lambda i: (i, 0)),
            pl.BlockSpec((N, D), lambda i: (0, 0)),
        ],
        out_specs=[
            pl.BlockSpec((B3, D), lambda i: (i, 0)),
            pl.BlockSpec((2, D), lambda i: (0, 0)),
        ],
        out_shape=[
            jax.ShapeDtypeStruct((N, D), jnp.float32),
            jax.ShapeDtypeStruct((2, D), jnp.float32),
        ],
        scratch_shapes=[pltpu.VMEM((2, D), jnp.float32)],
        compiler_params=pltpu.CompilerParams(
            dimension_semantics=("arbitrary",),
            vmem_limit_bytes=110 * 1024 * 1024,
        ),
    )(adj, support)

    out = pl.pallas_call(
        functools.partial(_bn_kernel, n_rows=float(N)),
        grid=(N // B1,),
        in_specs=[
            pl.BlockSpec((B1, D), lambda i: (i, 0)),
            pl.BlockSpec((2, D), lambda i: (0, 0)),
            pl.BlockSpec((1, D), lambda i: (0, 0)),
            pl.BlockSpec((1, D), lambda i: (0, 0)),
        ],
        out_specs=pl.BlockSpec((B1, D), lambda i: (i, 0)),
        out_shape=jax.ShapeDtypeStruct((N, D), jnp.float32),
    )(x_gcn, stats_loc, g_loc, b_loc)

    return out


# speedup vs baseline: 1.0825x; 1.0825x over previous
import jax
import jax.numpy as jnp
from jax.experimental import pallas as pl
from jax.experimental.pallas import tpu as pltpu

_EPS = 1e-5


def _emb_support_kernel(x_ref, wemb_ref, gamma_ref, beta_ref, wgcn_ref,
                        sup_ref, xemb_ref, acc_ref, *, n_blocks, block):
    i = pl.program_id(0)
    n_rows = float(n_blocks * block)

    @pl.when(i == 0)
    def _():
        acc_ref[...] = jnp.zeros_like(acc_ref)

    @pl.when(i < n_blocks)
    def _():
        xe = jnp.dot(x_ref[...], wemb_ref[...],
                     preferred_element_type=jnp.float32)
        j = jnp.minimum(i, n_blocks - 1)
        xemb_ref[pl.ds(j * block, block), :] = xe
        acc_ref[0:1, :] += jnp.sum(xe, axis=0, keepdims=True)
        acc_ref[1:2, :] += jnp.sum(xe * xe, axis=0, keepdims=True)

    @pl.when(i >= n_blocks)
    def _():
        mu = acc_ref[0:1, :] / n_rows
        var = acc_ref[1:2, :] / n_rows - mu * mu
        a = gamma_ref[...] * jax.lax.rsqrt(var + _EPS)
        b = beta_ref[...] - mu * a
        j = jnp.maximum(i - n_blocks, 0)
        h = jnp.maximum(xemb_ref[pl.ds(j * block, block), :] * a + b, 0.0)
        sup_ref[...] = jnp.dot(
            h, wgcn_ref[...], preferred_element_type=jnp.float32
        ).astype(jnp.bfloat16)


def _spmm_bn_kernel_gb(adj_ref, sup_ref, gamma_ref, beta_ref, out_ref,
                       xgcn_ref, acc_ref, *, n_blocks, block):
    i = pl.program_id(0)
    n_rows = float(n_blocks * block)

    @pl.when(i == 0)
    def _():
        acc_ref[...] = jnp.zeros_like(acc_ref)

    @pl.when(i < n_blocks)
    def _():
        a = adj_ref[...].astype(jnp.bfloat16)
        xg = jnp.dot(a, sup_ref[...], preferred_element_type=jnp.float32)
        j = jnp.minimum(i, n_blocks - 1)
        xgcn_ref[pl.ds(j * block, block), :] = xg
        acc_ref[0:1, :] += jnp.sum(xg, axis=0, keepdims=True)
        acc_ref[1:2, :] += jnp.sum(xg * xg, axis=0, keepdims=True)

    @pl.when(i >= n_blocks)
    def _():
        mu = acc_ref[0:1, :] / n_rows
        var = acc_ref[1:2, :] / n_rows - mu * mu
        a2 = gamma_ref[...] * jax.lax.rsqrt(var + _EPS)
        b2 = beta_ref[...] - mu * a2
        j = jnp.maximum(i - n_blocks, 0)
        out_ref[...] = xgcn_ref[pl.ds(j * block, block), :] * a2 + b2


import functools


def kernel(x, adj, W_emb, gcn_weight, gamma_in, beta_in, gamma_local,
           beta_local):
    N, F = x.shape
    D = W_emb.shape[1]
    g_in = gamma_in.reshape(1, D)
    b_in = beta_in.reshape(1, D)
    g_loc = gamma_local.reshape(1, D)
    b_loc = beta_local.reshape(1, D)

    B1 = 1000
    NB1 = N // B1
    support = pl.pallas_call(
        functools.partial(_emb_support_kernel, n_blocks=NB1, block=B1),
        grid=(2 * NB1,),
        in_specs=[
            pl.BlockSpec((B1, F), lambda i: (jnp.minimum(i, NB1 - 1), 0)),
            pl.BlockSpec((F, D), lambda i: (0, 0)),
            pl.BlockSpec((1, D), lambda i: (0, 0)),
            pl.BlockSpec((1, D), lambda i: (0, 0)),
            pl.BlockSpec((D, D), lambda i: (0, 0)),
        ],
        out_specs=pl.BlockSpec((B1, D), lambda i: (jnp.maximum(i - NB1, 0), 0)),
        out_shape=jax.ShapeDtypeStruct((N, D), jnp.bfloat16),
        scratch_shapes=[
            pltpu.VMEM((N, D), jnp.float32),
            pltpu.VMEM((2, D), jnp.float32),
        ],
        compiler_params=pltpu.CompilerParams(
            dimension_semantics=("arbitrary",)),
    )(x, W_emb, g_in, b_in, gcn_weight)

    B3 = 400
    NB3 = N // B3
    out = pl.pallas_call(
        functools.partial(_spmm_bn_kernel_gb, n_blocks=NB3, block=B3),
        grid=(2 * NB3,),
        in_specs=[
            pl.BlockSpec((B3, N), lambda i: (jnp.minimum(i, NB3 - 1), 0)),
            pl.BlockSpec((N, D), lambda i: (0, 0)),
            pl.BlockSpec((1, D), lambda i: (0, 0)),
            pl.BlockSpec((1, D), lambda i: (0, 0)),
        ],
        out_specs=pl.BlockSpec((B3, D), lambda i: (jnp.maximum(i - NB3, 0), 0)),
        out_shape=jax.ShapeDtypeStruct((N, D), jnp.float32),
        scratch_shapes=[
            pltpu.VMEM((N, D), jnp.float32),
            pltpu.VMEM((2, D), jnp.float32),
        ],
        compiler_params=pltpu.CompilerParams(
            dimension_semantics=("arbitrary",),
            vmem_limit_bytes=120 * 1024 * 1024,
        ),
    )(adj, support, g_loc, b_loc)

    return out


# speedup vs baseline: 1.1000x; 1.0162x over previous
import functools

import jax
import jax.numpy as jnp
from jax.experimental import pallas as pl
from jax.experimental.pallas import tpu as pltpu

_EPS = 1e-5


def _fused_kernel(x_ref, wemb_ref, g_in_ref, b_in_ref, wgcn_ref, adj_ref,
                  g_loc_ref, b_loc_ref, out_ref, xemb_ref, sup_ref, xgcn_ref,
                  acc_ref, *, nb1, b1, nb3, b3):
    i = pl.program_id(0)
    n_rows = float(nb1 * b1)
    p1, p2, p3 = nb1, 2 * nb1, 2 * nb1 + nb3

    @pl.when(i == 0)
    def _():
        acc_ref[...] = jnp.zeros_like(acc_ref)

    @pl.when(i < p1)
    def _():
        xe = jnp.dot(x_ref[...], wemb_ref[...],
                     preferred_element_type=jnp.float32)
        j = jnp.minimum(i, nb1 - 1)
        xemb_ref[pl.ds(j * b1, b1), :] = xe
        acc_ref[0:1, :] += jnp.sum(xe, axis=0, keepdims=True)
        acc_ref[1:2, :] += jnp.sum(xe * xe, axis=0, keepdims=True)

    @pl.when(jnp.logical_and(i >= p1, i < p2))
    def _():
        mu = acc_ref[0:1, :] / n_rows
        var = acc_ref[1:2, :] / n_rows - mu * mu
        a = g_in_ref[...] * jax.lax.rsqrt(var + _EPS)
        b = b_in_ref[...] - mu * a
        j = jnp.clip(i - p1, 0, nb1 - 1)
        h = jnp.maximum(xemb_ref[pl.ds(j * b1, b1), :] * a + b, 0.0)
        sup_ref[pl.ds(j * b1, b1), :] = jnp.dot(
            h, wgcn_ref[...], preferred_element_type=jnp.float32
        ).astype(jnp.bfloat16)

    @pl.when(jnp.logical_and(i >= p2, i < p3))
    def _():
        @pl.when(i == p2)
        def _():
            acc_ref[...] = jnp.zeros_like(acc_ref)

        a = adj_ref[...].astype(jnp.bfloat16)
        xg = jnp.dot(a, sup_ref[...], preferred_element_type=jnp.float32)
        j = jnp.clip(i - p2, 0, nb3 - 1)
        xgcn_ref[pl.ds(j * b3, b3), :] = xg
        acc_ref[0:1, :] += jnp.sum(xg, axis=0, keepdims=True)
        acc_ref[1:2, :] += jnp.sum(xg * xg, axis=0, keepdims=True)

    @pl.when(i >= p3)
    def _():
        mu = acc_ref[0:1, :] / n_rows
        var = acc_ref[1:2, :] / n_rows - mu * mu
        a2 = g_loc_ref[...] * jax.lax.rsqrt(var + _EPS)
        b2 = b_loc_ref[...] - mu * a2
        j = jnp.clip(i - p3, 0, nb3 - 1)
        out_ref[...] = xgcn_ref[pl.ds(j * b3, b3), :] * a2 + b2


def kernel(x, adj, W_emb, gcn_weight, gamma_in, beta_in, gamma_local,
           beta_local):
    N, F = x.shape
    D = W_emb.shape[1]
    g_in = gamma_in.reshape(1, D)
    b_in = beta_in.reshape(1, D)
    g_loc = gamma_local.reshape(1, D)
    b_loc = beta_local.reshape(1, D)

    B1 = 1000
    NB1 = N // B1
    B3 = 400
    NB3 = N // B3
    p2, p3 = 2 * NB1, 2 * NB1 + NB3
    grid = 2 * NB1 + 2 * NB3

    out = pl.pallas_call(
        functools.partial(_fused_kernel, nb1=NB1, b1=B1, nb3=NB3, b3=B3),
        grid=(grid,),
        in_specs=[
            pl.BlockSpec((B1, F), lambda i: (jnp.minimum(i, NB1 - 1), 0)),
            pl.BlockSpec((F, D), lambda i: (0, 0)),
            pl.BlockSpec((1, D), lambda i: (0, 0)),
            pl.BlockSpec((1, D), lambda i: (0, 0)),
            pl.BlockSpec((D, D), lambda i: (0, 0)),
            pl.BlockSpec((B3, N), lambda i: (jnp.clip(i - p2, 0, NB3 - 1), 0)),
            pl.BlockSpec((1, D), lambda i: (0, 0)),
            pl.BlockSpec((1, D), lambda i: (0, 0)),
        ],
        out_specs=pl.BlockSpec((B3, D), lambda i: (jnp.clip(i - p3, 0, NB3 - 1), 0)),
        out_shape=jax.ShapeDtypeStruct((N, D), jnp.float32),
        scratch_shapes=[
            pltpu.VMEM((N, D), jnp.float32),
            pltpu.VMEM((N, D), jnp.bfloat16),
            pltpu.VMEM((N, D), jnp.float32),
            pltpu.VMEM((2, D), jnp.float32),
        ],
        compiler_params=pltpu.CompilerParams(
            dimension_semantics=("arbitrary",),
            vmem_limit_bytes=120 * 1024 * 1024,
        ),
    )(x, W_emb, g_in, b_in, gcn_weight, adj, g_loc, b_loc)

    return out
